# baseline (speedup 1.0000x reference)
import functools

import jax
import jax.numpy as jnp
from jax import lax
from jax.experimental import pallas as pl
from jax.experimental.pallas import tpu as pltpu
from jax.experimental.pallas import tpu_sc as plsc

_B = 16384
_D = 64
_NC = 2
_NS = 16
_NW = _NC * _NS
_BPW = _B // _NW
_L = 16
_TPAD = _L + 1


def _tt_kernel(user_idx, item_idx, user_table, item_table, out_hbm,
               uidx_v, iidx_v, urows_v, irows_v, out_v, tbuf_v, sem_u, sem_i):
    wid = lax.axis_index("s") * _NC + lax.axis_index("c")
    base = wid * _BPW

    pltpu.sync_copy(user_idx.at[pl.ds(base, _BPW)], uidx_v)
    pltpu.sync_copy(item_idx.at[pl.ds(base, _BPW)], iidx_v)

    cu = pltpu.async_copy(user_table.at[uidx_v], urows_v, sem_u)
    ci = pltpu.async_copy(item_table.at[iidx_v], irows_v, sem_i)
    cu.wait()
    ci.wait()

    lane_iota = lax.iota(jnp.int32, _L)

    def blk_body(blk, carry):
        r0 = blk * _L
        acc = jnp.zeros((_L,), jnp.float32)
        for j in range(_L):
            row = r0 + j
            pu = (urows_v[row, pl.ds(0, _L)] * irows_v[row, pl.ds(0, _L)]
                  + urows_v[row, pl.ds(_L, _L)] * irows_v[row, pl.ds(_L, _L)]
                  + urows_v[row, pl.ds(2 * _L, _L)] * irows_v[row, pl.ds(2 * _L, _L)]
                  + urows_v[row, pl.ds(3 * _L, _L)] * irows_v[row, pl.ds(3 * _L, _L)])
            acc = jnp.where(lane_iota == j, jnp.sum(pu), acc)
        out_v[pl.ds(r0, _L)] = acc
        return carry

    lax.fori_loop(0, _BPW // _L, blk_body, 0)

    pltpu.sync_copy(out_v, out_hbm.at[pl.ds(base, _BPW)])


@jax.jit
def kernel(user_idx, item_idx, user_table, item_table):
    mesh = plsc.VectorSubcoreMesh(core_axis_name="c", subcore_axis_name="s")
    f = functools.partial(
        pl.kernel,
        out_type=jax.ShapeDtypeStruct((_B,), jnp.float32),
        mesh=mesh,
        compiler_params=pltpu.CompilerParams(needs_layout_passes=False,
                                             use_tc_tiling_on_sc=False),
        scratch_types=[
            pltpu.VMEM((_BPW,), jnp.int32),
            pltpu.VMEM((_BPW,), jnp.int32),
            pltpu.VMEM((_BPW, _D), jnp.float32),
            pltpu.VMEM((_BPW, _D), jnp.float32),
            pltpu.VMEM((_BPW,), jnp.float32),
            pltpu.VMEM((_L * _TPAD,), jnp.float32),
            pltpu.SemaphoreType.DMA,
            pltpu.SemaphoreType.DMA,
        ],
    )(_tt_kernel)
    return f(user_idx.astype(jnp.int32), item_idx.astype(jnp.int32),
             user_table, item_table)
